# baseline (speedup 1.0000x reference)
import jax
import jax.numpy as jnp
from jax import lax
from jax.experimental import pallas as pl
from jax.experimental.pallas import tpu as pltpu
from jax.experimental.pallas import tpu_sc as plsc

_N_COLS = 26
_VOCAB = 100000
_D = 32
_B = 16384
_BT = _B * _N_COLS
_NC, _NS = 2, 16
_NW = _NC * _NS
_PW = _BT // _NW
_CH = 128
_NCH = _PW // _CH
_K = 8
_LANES = 16


def _body(xf, table, out, idx_v, rows_v, *gsems):
    wid = lax.axis_index("s") * _NC + lax.axis_index("c")
    base = wid * _PW

    pltpu.sync_copy(xf.at[pl.ds(base, _PW)], idx_v)

    lanes = lax.iota(jnp.int32, _LANES)
    @pl.loop(0, _PW // _LANES, unroll=8)
    def _add(i):
        sl = pl.ds(i * _LANES, _LANES)
        pos = (base + i * _LANES) + lanes
        idx_v[sl] = idx_v[sl] + lax.rem(pos, _N_COLS) * _VOCAB

    def _gather(g, slot):
        src = table.at[idx_v.at[pl.ds(pl.multiple_of(g * _CH, _CH), _CH)]]
        pltpu.async_copy(src, rows_v.at[slot], gsems[slot])

    for b in range(_K):
        _gather(b, b)

    @pl.loop(0, _NCH, step=_K)
    def _main(g0):
        for b in range(_K):
            g = g0 + b
            pltpu.make_async_copy(
                table.at[idx_v.at[pl.ds(pl.multiple_of(g * _CH, _CH), _CH)]],
                rows_v.at[b], gsems[b]).wait()
            dst = out.at[pl.ds(pl.multiple_of(base + g * _CH, _CH), _CH)]
            pltpu.sync_copy(rows_v.at[b], dst)

            @pl.when(g + _K < _NCH)
            def _():
                _gather(g + _K, b)


def kernel(X, table):
    xf = X.reshape(_BT)
    mesh = plsc.VectorSubcoreMesh(
        core_axis_name="c", subcore_axis_name="s",
        num_cores=_NC, num_subcores=_NS)
    scratch = [
        pltpu.VMEM((_PW,), jnp.int32),
        pltpu.VMEM((_K, _CH, _D), jnp.float32),
    ] + [pltpu.SemaphoreType.DMA] * _K
    out = pl.kernel(
        _body,
        out_type=jax.ShapeDtypeStruct((_BT, _D), jnp.float32),
        mesh=mesh,
        scratch_types=scratch,
        compiler_params=pltpu.CompilerParams(use_tc_tiling_on_sc=False),
    )(xf, table)
    return out.reshape(_B, _N_COLS, _D)


# speedup vs baseline: 1.2471x; 1.2471x over previous
import jax
import jax.numpy as jnp
from jax import lax
from jax.experimental import pallas as pl
from jax.experimental.pallas import tpu as pltpu
from jax.experimental.pallas import tpu_sc as plsc

_N_COLS = 26
_VOCAB = 100000
_D = 32
_B = 16384
_BT = _B * _N_COLS
_NC, _NS = 2, 16
_NW = _NC * _NS
_PW = _BT // _NW
_CH = 256
_NCH = _PW // _CH
_LANES = 16


def _body(xf, table, out, idx_v, rows_v, gsem, osem):
    wid = lax.axis_index("s") * _NC + lax.axis_index("c")
    base = wid * _PW
    lanes = lax.iota(jnp.int32, _LANES)

    @pl.loop(0, _NCH)
    def _block(g):
        start = base + g * _CH
        pltpu.sync_copy(xf.at[pl.ds(start, _CH)], idx_v)

        buf = lax.rem(g, 2)

        @pl.loop(0, _CH // _LANES)
        def _vec(k):
            v = idx_v[pl.ds(k * _LANES, _LANES)]
            rv = v + lax.rem((start + k * _LANES) + lanes, _N_COLS) * _VOCAB
            for lane in range(_LANES):
                pltpu.async_copy(
                    table.at[rv[lane]], rows_v.at[buf, k * _LANES + lane], gsem)

        @pl.loop(0, _CH)
        def _drain(i):
            pltpu.make_async_copy(table.at[0], rows_v.at[buf, i], gsem).wait()

        @pl.when(g > 0)
        def _():
            pltpu.make_async_copy(
                rows_v.at[1 - buf], out.at[pl.ds(0, _CH)], osem).wait()

        pltpu.async_copy(rows_v.at[buf], out.at[pl.ds(start, _CH)], osem)

    pltpu.make_async_copy(
        rows_v.at[lax.rem(_NCH - 1, 2)],
        out.at[pl.ds(0, _CH)], osem).wait()


def kernel(X, table):
    xf = X.reshape(_BT)
    mesh = plsc.VectorSubcoreMesh(
        core_axis_name="c", subcore_axis_name="s",
        num_cores=_NC, num_subcores=_NS)
    scratch = [
        pltpu.VMEM((_CH,), jnp.int32),
        pltpu.VMEM((2, _CH, _D), jnp.float32),
        pltpu.SemaphoreType.DMA,
        pltpu.SemaphoreType.DMA,
    ]
    out = pl.kernel(
        _body,
        out_type=jax.ShapeDtypeStruct((_BT, _D), jnp.float32),
        mesh=mesh,
        scratch_types=scratch,
    )(xf, table)
    return out.reshape(_B, _N_COLS, _D)


# speedup vs baseline: 1.2969x; 1.0400x over previous
import jax
import jax.numpy as jnp
from jax import lax
from jax.experimental import pallas as pl
from jax.experimental.pallas import tpu as pltpu
from jax.experimental.pallas import tpu_sc as plsc

_N_COLS = 26
_VOCAB = 100000
_D = 32
_B = 16384
_BT = _B * _N_COLS
_NC, _NS = 2, 16
_NW = _NC * _NS
_BPW = _B // _NW
_NBB = 4
_CH = _NBB * _N_COLS
_NBLK = _BPW // _NBB
_LANES = 16
_NVEC = (_CH + _LANES - 1) // _LANES


def _body(xf, table, out, idx_v, rows_v, gsem, osem):
    wid = lax.axis_index("s") * _NC + lax.axis_index("c")
    base = wid * _BPW * _N_COLS
    lanes = lax.iota(jnp.int32, _LANES)

    @pl.loop(0, _NBLK)
    def _block(g):
        start = base + g * _CH
        b0 = wid * _BPW + g * _NBB
        pltpu.sync_copy(xf.at[pl.ds(start, _CH)], idx_v.at[pl.ds(0, _CH)])

        buf = lax.rem(g, 2)

        for k in range(_NVEC):
            v = idx_v[pl.ds(k * _LANES, _LANES)]
            rv = v + lax.rem(lanes + (k * _LANES), _N_COLS) * _VOCAB
            nlive = min(_LANES, _CH - k * _LANES)
            for lane in range(nlive):
                i = k * _LANES + lane
                pltpu.async_copy(
                    table.at[rv[lane]], rows_v.at[buf, i // _N_COLS, i % _N_COLS],
                    gsem)

        for i in range(_CH):
            pltpu.make_async_copy(
                table.at[0], rows_v.at[buf, i // _N_COLS, i % _N_COLS],
                gsem).wait()

        @pl.when(g > 0)
        def _():
            pltpu.make_async_copy(
                rows_v.at[1 - buf], out.at[pl.ds(0, _NBB)], osem).wait()

        pltpu.async_copy(rows_v.at[buf], out.at[pl.ds(b0, _NBB)], osem)

    pltpu.make_async_copy(
        rows_v.at[lax.rem(_NBLK - 1, 2)],
        out.at[pl.ds(0, _NBB)], osem).wait()


def kernel(X, table):
    xf = X.reshape(_BT)
    mesh = plsc.VectorSubcoreMesh(
        core_axis_name="c", subcore_axis_name="s",
        num_cores=_NC, num_subcores=_NS)
    scratch = [
        pltpu.VMEM((_NVEC * _LANES,), jnp.int32),
        pltpu.VMEM((2, _NBB, _N_COLS, _D), jnp.float32),
        pltpu.SemaphoreType.DMA,
        pltpu.SemaphoreType.DMA,
    ]
    return pl.kernel(
        _body,
        out_type=jax.ShapeDtypeStruct((_B, _N_COLS, _D), jnp.float32),
        mesh=mesh,
        scratch_types=scratch,
    )(xf, table)


# speedup vs baseline: 1.3173x; 1.0157x over previous
import jax
import jax.numpy as jnp
from jax import lax
from jax.experimental import pallas as pl
from jax.experimental.pallas import tpu as pltpu
from jax.experimental.pallas import tpu_sc as plsc

_N_COLS = 26
_VOCAB = 100000
_D = 32
_B = 16384
_NC, _NS = 2, 16
_NW = _NC * _NS
_BPW = _B // _NW
_NBB = 4
_NSUP = _BPW // (2 * _NBB)
_LANES = 16


def _body(X, table, out, idx1, rows_v, gsem, isem, osem0, osem1):
    wid = lax.axis_index("s") * _NC + lax.axis_index("c")
    lanes = lax.iota(jnp.int32, _LANES)
    off_lo = lanes * _VOCAB
    off_hi = (lanes + _LANES) * _VOCAB
    osems = (osem0, osem1)

    @pl.loop(0, _NSUP)
    def _super(g):
        b0 = wid * _BPW + g * (2 * _NBB)
        for bl in range(2 * _NBB):
            pltpu.async_copy(X.at[b0 + bl], idx1.at[bl], isem)
        for bl in range(2 * _NBB):
            pltpu.make_async_copy(X.at[0], idx1.at[bl], isem).wait()

        for half in range(2):
            bh = b0 + half * _NBB

            for bl in range(_NBB):
                row = half * _NBB + bl
                rv0 = plsc.bitcast(
                    idx1[row, pl.ds(0, _LANES)], jnp.int32) + off_lo
                rv1 = plsc.bitcast(
                    idx1[row, pl.ds(_LANES, _LANES)], jnp.int32) + off_hi
                for lane in range(_LANES):
                    pltpu.async_copy(
                        table.at[rv0[lane]], rows_v.at[half, bl, lane], gsem)
                for lane in range(_N_COLS - _LANES):
                    pltpu.async_copy(
                        table.at[rv1[lane]], rows_v.at[half, bl, _LANES + lane],
                        gsem)

            for bl in range(_NBB):
                for c in range(_N_COLS):
                    pltpu.make_async_copy(
                        table.at[0], rows_v.at[half, bl, c], gsem).wait()

            @pl.when(g > 0)
            def _():
                pltpu.make_async_copy(
                    rows_v.at[half], out.at[pl.ds(0, _NBB)], osems[half]).wait()

            pltpu.async_copy(rows_v.at[half], out.at[pl.ds(bh, _NBB)],
                             osems[half])

    for half in range(2):
        pltpu.make_async_copy(
            rows_v.at[half], out.at[pl.ds(0, _NBB)], osems[half]).wait()


def kernel(X, table):
    Xp = jax.lax.bitcast_convert_type(
        jnp.pad(X, ((0, 0), (0, 32 - _N_COLS))), jnp.float32)
    mesh = plsc.VectorSubcoreMesh(
        core_axis_name="c", subcore_axis_name="s",
        num_cores=_NC, num_subcores=_NS)
    scratch = [
        pltpu.VMEM((2 * _NBB, _D), jnp.float32),
        pltpu.VMEM((2, _NBB, _N_COLS, _D), jnp.float32),
        pltpu.SemaphoreType.DMA,
        pltpu.SemaphoreType.DMA,
        pltpu.SemaphoreType.DMA,
        pltpu.SemaphoreType.DMA,
    ]
    return pl.kernel(
        _body,
        out_type=jax.ShapeDtypeStruct((_B, _N_COLS, _D), jnp.float32),
        mesh=mesh,
        scratch_types=scratch,
        compiler_params=pltpu.CompilerParams(needs_layout_passes=False),
    )(Xp, table)
